# baseline (speedup 1.0000x reference)
import functools

import jax
import jax.numpy as jnp
from jax import lax
from jax.experimental import pallas as pl
from jax.experimental.pallas import tpu as pltpu
from jax.experimental.pallas import tpu_sc as plsc

D = 128
NC = 2
NS = 16
NW = NC * NS
A = 4096
S = 50
SP = 56
CHUNK = 128
NBUF = 4
NH = 2
AH = A // NH

_mesh = plsc.VectorSubcoreMesh(core_axis_name="c", subcore_axis_name="s")


def _make_embed(a_rows):
    n_ch = a_rows * SP // (NW * CHUNK)

    @functools.partial(
        pl.kernel,
        out_type=jax.ShapeDtypeStruct((a_rows * SP, D), jnp.float32),
        mesh=_mesh,
        scratch_types=[
            pltpu.VMEM((n_ch, CHUNK), jnp.int32),
            pltpu.VMEM((NBUF, CHUNK, D), jnp.float32),
            pltpu.SemaphoreType.DMA,
            pltpu.SemaphoreType.DMA,
        ],
    )
    def _embed(idx_hbm, table_hbm, out_hbm, idx_v, rows_v, gsem, wsem):
        wid = lax.axis_index("s") * NC + lax.axis_index("c")
        base = wid * n_ch * CHUNK
        pltpu.sync_copy(idx_hbm.at[wid], idx_v)

        def gather(j, b):
            pltpu.async_copy(table_hbm.at[idx_v.at[j]], rows_v.at[b], gsem)

        def wb(j, b):
            pltpu.async_copy(
                rows_v.at[b], out_hbm.at[pl.ds(base + j * CHUNK, CHUNK)], wsem
            )

        def wait_gather(b):
            pltpu.make_async_copy(
                table_hbm.at[pl.ds(0, CHUNK)], rows_v.at[b], gsem
            ).wait()

        def wait_wb(b):
            pltpu.make_async_copy(
                rows_v.at[b], out_hbm.at[pl.ds(base, CHUNK)], wsem
            ).wait()

        for k in range(NBUF - 1):
            gather(k, k)

        @pl.loop(0, n_ch)
        def _(j):
            b = lax.rem(j, NBUF)
            wait_gather(b)
            wb(j, b)
            @pl.when(jnp.logical_and(j > 0, j < n_ch - (NBUF - 1)))
            def _():
                wait_wb(b)

            @pl.when(j < n_ch - (NBUF - 1))
            def _():
                gather(j + NBUF - 1, lax.rem(j + NBUF - 1, NBUF))

        for _k in range(NBUF):
            wait_wb(0)

    return _embed


_embed_half = _make_embed(AH)


def kernel(x, table):
    pad_idx = jnp.arange(A * (SP - S), dtype=jnp.int32).reshape(A, SP - S)
    outs = []
    for h in range(NH):
        xh = lax.slice_in_dim(x, h * AH, (h + 1) * AH, axis=0)
        ph = lax.slice_in_dim(pad_idx, h * AH, (h + 1) * AH, axis=0)
        idx = jnp.concatenate([xh, ph], axis=1).reshape(NW, -1, CHUNK)
        out = _embed_half(idx, table)
        o3 = out.reshape(AH, SP, D)[:, :S, :]
        outs.append(jnp.where(xh[:, :, None] >= 0, o3, 0.0))
    return jnp.concatenate(outs, axis=0)


# speedup vs baseline: 1.0786x; 1.0786x over previous
import functools

import jax
import jax.numpy as jnp
from jax import lax
from jax.experimental import pallas as pl
from jax.experimental.pallas import tpu as pltpu
from jax.experimental.pallas import tpu_sc as plsc

D = 128
NC = 2
NS = 16
NW = NC * NS
A = 4096
S = 50
SP = 56
CHUNK = 128
NBUF = 4
NH = 2
AH = A // NH

_mesh = plsc.VectorSubcoreMesh(core_axis_name="c", subcore_axis_name="s")


def _make_embed(a_rows):
    n_ch = a_rows * SP // (NW * CHUNK)

    @functools.partial(
        pl.kernel,
        out_type=jax.ShapeDtypeStruct((a_rows * SP, D), jnp.float32),
        mesh=_mesh,
        scratch_types=[
            pltpu.VMEM((n_ch, CHUNK), jnp.int32),
            pltpu.VMEM((NBUF, CHUNK, D), jnp.float32),
            pltpu.SemaphoreType.DMA,
            pltpu.SemaphoreType.DMA,
        ],
    )
    def _embed(idx_hbm, table_hbm, out_hbm, idx_v, rows_v, gsem, wsem):
        wid = lax.axis_index("s") * NC + lax.axis_index("c")
        base = wid * n_ch * CHUNK
        pltpu.sync_copy(idx_hbm.at[wid], idx_v)

        def gather(j, b):
            pltpu.async_copy(table_hbm.at[idx_v.at[j]], rows_v.at[b], gsem)

        def wb(j, b):
            pltpu.async_copy(
                rows_v.at[b], out_hbm.at[pl.ds(base + j * CHUNK, CHUNK)], wsem
            )

        def wait_gather(b):
            pltpu.make_async_copy(
                table_hbm.at[pl.ds(0, CHUNK)], rows_v.at[b], gsem
            ).wait()

        def wait_wb(b):
            pltpu.make_async_copy(
                rows_v.at[b], out_hbm.at[pl.ds(base, CHUNK)], wsem
            ).wait()

        for k in range(NBUF - 1):
            gather(k, k)

        @pl.loop(0, n_ch)
        def _(j):
            b = lax.rem(j, NBUF)
            wait_gather(b)
            wb(j, b)
            @pl.when(jnp.logical_and(j > 0, j < n_ch - (NBUF - 1)))
            def _():
                wait_wb(b)

            @pl.when(j < n_ch - (NBUF - 1))
            def _():
                gather(j + NBUF - 1, lax.rem(j + NBUF - 1, NBUF))

        for _k in range(NBUF):
            wait_wb(0)

    return _embed


_embed_half = _make_embed(AH)


def kernel(x, table):
    pad_idx = jnp.arange(A * (SP - S), dtype=jnp.int32).reshape(A, SP - S)
    outf = jnp.empty((A, S, D), jnp.float32)
    for h in range(NH):
        xh = lax.slice_in_dim(x, h * AH, (h + 1) * AH, axis=0)
        ph = lax.slice_in_dim(pad_idx, h * AH, (h + 1) * AH, axis=0)
        idx = jnp.concatenate([xh, ph], axis=1).reshape(NW, -1, CHUNK)
        out = _embed_half(idx, table)
        o3 = out.reshape(AH, SP, D)[:, :S, :]
        oh = jnp.where(xh[:, :, None] >= 0, o3, 0.0)
        outf = lax.dynamic_update_slice(outf, oh, (h * AH, 0, 0))
    return outf


# speedup vs baseline: 1.3756x; 1.2754x over previous
import functools

import jax
import jax.numpy as jnp
from jax import lax
from jax.experimental import pallas as pl
from jax.experimental.pallas import tpu as pltpu
from jax.experimental.pallas import tpu_sc as plsc

D = 128
NC = 2
NS = 16
NW = NC * NS
A = 4096
S = 50
SP = 56
CHUNK = 128
N_CH = A * SP // (NW * CHUNK)
NBUF = 6

_mesh = plsc.VectorSubcoreMesh(core_axis_name="c", subcore_axis_name="s")


@functools.partial(
    pl.kernel,
    out_type=jax.ShapeDtypeStruct((A * SP, D), jnp.float32),
    mesh=_mesh,
    scratch_types=[
        pltpu.VMEM((N_CH, CHUNK), jnp.int32),
        pltpu.VMEM((NBUF, CHUNK, D), jnp.float32),
        pltpu.SemaphoreType.DMA,
        pltpu.SemaphoreType.DMA,
    ],
)
def _embed(idx_hbm, table_hbm, out_hbm, idx_v, rows_v, gsem, wsem):
    wid = lax.axis_index("s") * NC + lax.axis_index("c")
    base = wid * N_CH * CHUNK
    pltpu.sync_copy(idx_hbm.at[wid], idx_v)

    def gather(j, b):
        pltpu.async_copy(table_hbm.at[idx_v.at[j]], rows_v.at[b], gsem)

    def wb(j, b):
        pltpu.async_copy(
            rows_v.at[b], out_hbm.at[pl.ds(base + j * CHUNK, CHUNK)], wsem
        )

    def wait_gather(b):
        pltpu.make_async_copy(
            table_hbm.at[pl.ds(0, CHUNK)], rows_v.at[b], gsem
        ).wait()

    def wait_wb(b):
        pltpu.make_async_copy(
            rows_v.at[b], out_hbm.at[pl.ds(base, CHUNK)], wsem
        ).wait()

    for k in range(NBUF - 1):
        gather(k, k)

    @pl.loop(0, N_CH)
    def _(j):
        b = lax.rem(j, NBUF)
        wait_gather(b)
        wb(j, b)
        @pl.when(jnp.logical_and(j > 0, j < N_CH - (NBUF - 1)))
        def _():
            wait_wb(b)

        @pl.when(j < N_CH - (NBUF - 1))
        def _():
            gather(j + NBUF - 1, lax.rem(j + NBUF - 1, NBUF))

    for _k in range(NBUF):
        wait_wb(0)


def kernel(x, table):
    pad_idx = jnp.arange(A * (SP - S), dtype=jnp.int32).reshape(A, SP - S)
    idx = jnp.concatenate([x, pad_idx], axis=1).reshape(NW, N_CH, CHUNK)
    out = _embed(idx, table)
    out3 = out.reshape(A, SP, D)[:, :S, :]
    return jnp.where(x[:, :, None] >= 0, out3, 0.0)
